# Initial kernel scaffold; baseline (speedup 1.0000x reference)
import jax
import jax.numpy as jnp
from jax.experimental import pallas as pl


def kernel(edge_index, emb, W1, b1, W2, b2):
    raise NotImplementedError("write your pallas kernel here")


# speedup vs baseline: 10.5504x; 10.5504x over previous
import functools

import jax
import jax.numpy as jnp
from jax import lax
from jax.experimental import pallas as pl
from jax.experimental.pallas import tpu as pltpu
from jax.experimental.pallas import tpu_sc as plsc

N = 50000
E = 800000
DE = 64
DH = 128
DO = 64

NC = 2
NS = 16

RPC = 25088
NP = NC * RPC
RPT = RPC // NS
TRASH = RPC
ACC_ROWS = RPC + 128

ET = E // NS
CH = 2000
NCH = ET // CH
CBUF = 2048
GK = 128
ICH = 112

_sc_mesh = plsc.VectorSubcoreMesh(core_axis_name="c", subcore_axis_name="s")


def _deg_body(dst_hbm, deg_hbm, dst_v, idx2d, ones_v, zeros_v, acc):
    c = lax.axis_index("c")
    s = lax.axis_index("s")
    base = c * RPC

    def zb(j, _):
        zeros_v[pl.ds(j * 16, 16)] = jnp.zeros((16,), jnp.float32)
        return 0
    lax.fori_loop(0, RPT // 16, zb, 0)
    pltpu.sync_copy(zeros_v, acc.at[pl.ds(s * RPT, RPT)])

    def ob(j, _):
        ones_v[pl.ds(j * 16, 16)] = jnp.ones((16,), jnp.float32)
        return 0
    lax.fori_loop(0, GK // 16, ob, 0)
    for k in range(3):
        idx2d[15, pl.ds(80 + k * 16, 16)] = jnp.full((16,), TRASH, jnp.int32)

    plsc.subcore_barrier()

    def chunk(ci, _):
        off = s * ET + ci * CH
        pltpu.sync_copy(dst_hbm.at[pl.ds(off, CH)], dst_v)

        def build(j, _):
            dvec = dst_v[pl.ds(j * 16, 16)]
            dloc = dvec - base
            m = (dloc >= 0) & (dloc < RPC)
            dsel = jnp.where(m, dloc, TRASH)
            r = j >> 3
            col = (j & 7) * 16
            idx2d[r, pl.ds(col, 16)] = dsel
            return 0
        lax.fori_loop(0, CH // 16, build, 0)

        for j2 in range(CBUF // GK):
            pltpu.sync_copy(ones_v, acc.at[idx2d.at[j2]], add=True)
        return 0
    lax.fori_loop(0, NCH, chunk, 0)

    plsc.subcore_barrier()
    pltpu.sync_copy(acc.at[pl.ds(s * RPT, RPT)], zeros_v)
    pltpu.sync_copy(zeros_v, deg_hbm.at[pl.ds(base + s * RPT, RPT)])


_deg_call = pl.kernel(
    _deg_body,
    out_type=jax.ShapeDtypeStruct((NP,), jnp.float32),
    mesh=_sc_mesh,
    scratch_types=[
        pltpu.VMEM((CH,), jnp.int32),
        pltpu.VMEM((CBUF // GK, GK), jnp.int32),
        pltpu.VMEM((GK,), jnp.float32),
        pltpu.VMEM((RPT,), jnp.float32),
        pltpu.VMEM_SHARED((ACC_ROWS,), jnp.float32),
    ],
)


def _agg_body(g_hbm, src_hbm, dst_hbm, out_hbm,
              dst_v, src_v, comp_src, comp_dst, rows_v, stage_v, sem, acc):
    c = lax.axis_index("c")
    s = lax.axis_index("s")
    base = c * RPC
    r0 = s * RPT

    def init(k, _):
        pltpu.sync_copy(g_hbm.at[pl.ds(base + r0 + k * ICH, ICH)], stage_v)
        pltpu.sync_copy(stage_v, acc.at[pl.ds(r0 + k * ICH, ICH)])
        return 0
    lax.fori_loop(0, RPT // ICH, init, 0)
    plsc.subcore_barrier()

    def chunk(ci, _):
        off = s * ET + ci * CH
        pltpu.sync_copy(dst_hbm.at[pl.ds(off, CH)], dst_v)
        pltpu.sync_copy(src_hbm.at[pl.ds(off, CH)], src_v)

        def comp(j, cnt):
            dvec = dst_v[pl.ds(j * 16, 16)]
            svec = src_v[pl.ds(j * 16, 16)]
            dloc = dvec - base
            m = (dloc >= 0) & (dloc < RPC)
            incl = plsc.cumsum(jnp.where(m, 1, 0))
            p = cnt + incl - 1
            plsc.store_scatter(comp_src, [p], svec, mask=m)
            plsc.store_scatter(comp_dst, [p >> 7, p & 127], dloc, mask=m)
            return cnt + incl[15]
        m_cnt = lax.fori_loop(0, CH // 16, comp, jnp.int32(0))

        iota16 = lax.iota(jnp.int32, 16)
        mp = (m_cnt + (GK - 1)) & (-GK)
        for j in range(GK // 16):
            pos = m_cnt + j * 16 + iota16
            pm = pos < mp
            plsc.store_scatter(comp_src, [pos],
                               jnp.zeros((16,), jnp.int32), mask=pm)
            plsc.store_scatter(comp_dst, [pos >> 7, pos & 127],
                               jnp.full((16,), TRASH, jnp.int32), mask=pm)

        def gs(k, _):
            pltpu.async_copy(g_hbm.at[comp_src.at[pl.ds(k * GK, GK)]],
                             rows_v, sem).wait()
            pltpu.sync_copy(rows_v, acc.at[comp_dst.at[k]], add=True)
            return 0
        lax.fori_loop(0, mp >> 7, gs, 0)
        return 0
    lax.fori_loop(0, NCH, chunk, 0)

    plsc.subcore_barrier()

    def copyout(k, _):
        pltpu.sync_copy(acc.at[pl.ds(r0 + k * ICH, ICH)], stage_v)
        pltpu.sync_copy(stage_v, out_hbm.at[pl.ds(base + r0 + k * ICH, ICH)])
        return 0
    lax.fori_loop(0, RPT // ICH, copyout, 0)


_agg_call = pl.kernel(
    _agg_body,
    out_type=jax.ShapeDtypeStruct((NP, DE), jnp.float32),
    mesh=_sc_mesh,
    compiler_params=pltpu.CompilerParams(needs_layout_passes=False,
                                         use_tc_tiling_on_sc=False),
    scratch_types=[
        pltpu.VMEM((CH,), jnp.int32),
        pltpu.VMEM((CH,), jnp.int32),
        pltpu.VMEM((CBUF,), jnp.int32),
        pltpu.VMEM((CBUF // GK, GK), jnp.int32),
        pltpu.VMEM((GK, DE), jnp.float32),
        pltpu.VMEM((ICH, DE), jnp.float32),
        pltpu.SemaphoreType.DMA,
        pltpu.VMEM_SHARED((ACC_ROWS, DE), jnp.float32),
    ],
)


_TC_R = 1568
_TC_GRID = NP // _TC_R


def _prep_body(deg_ref, emb_ref, dis_ref, g_ref):
    dis = lax.rsqrt(deg_ref[...] + 1.0)
    dis_ref[...] = dis
    g_ref[...] = emb_ref[...] * dis


def _mid_body(s1_ref, dis_ref, w1_ref, b1_ref, w2_ref, g2_ref):
    dis = dis_ref[...]
    x = s1_ref[...] * dis
    h = jnp.dot(x, w1_ref[...], preferred_element_type=jnp.float32)
    h = jnp.maximum(h + b1_ref[...], 0.0)
    g2_ref[...] = jnp.dot(h, w2_ref[...],
                          preferred_element_type=jnp.float32) * dis


def _post_body(s2_ref, dis_ref, b2_ref, out_ref):
    out_ref[...] = s2_ref[...] * dis_ref[...] + b2_ref[...]


def _row_spec(d):
    return pl.BlockSpec((_TC_R, d), lambda i: (i, 0))


def _full_spec(r, d):
    return pl.BlockSpec((r, d), lambda i: (0, 0))


_prep_call = pl.pallas_call(
    _prep_body,
    grid=(_TC_GRID,),
    in_specs=[_row_spec(1), _row_spec(DE)],
    out_specs=[_row_spec(1), _row_spec(DE)],
    out_shape=[jax.ShapeDtypeStruct((NP, 1), jnp.float32),
               jax.ShapeDtypeStruct((NP, DE), jnp.float32)],
)

_mid_call = pl.pallas_call(
    _mid_body,
    grid=(_TC_GRID,),
    in_specs=[_row_spec(DE), _row_spec(1), _full_spec(DE, DH),
              _full_spec(1, DH), _full_spec(DH, DO)],
    out_specs=_row_spec(DO),
    out_shape=jax.ShapeDtypeStruct((NP, DO), jnp.float32),
)

_post_call = pl.pallas_call(
    _post_body,
    grid=(_TC_GRID,),
    in_specs=[_row_spec(DO), _row_spec(1), _full_spec(1, DO)],
    out_specs=_row_spec(DO),
    out_shape=jax.ShapeDtypeStruct((NP, DO), jnp.float32),
)


def kernel(edge_index, emb, W1, b1, W2, b2):
    src = edge_index[0]
    dst = edge_index[1]
    emb_pad = jnp.pad(emb, ((0, NP - N), (0, 0)))

    deg = _deg_call(dst)
    dis, g1 = _prep_call(deg.reshape(NP, 1), emb_pad)
    s1 = _agg_call(g1, src, dst)
    g2 = _mid_call(s1, dis, W1, b1.reshape(1, DH), W2)
    s2 = _agg_call(g2, src, dst)
    out_pad = _post_call(s2, dis, b2.reshape(1, DO))
    return out_pad[:N]
